# baseline (speedup 1.0000x reference)
import functools

import numpy as np

import jax
import jax.numpy as jnp
from jax.experimental import pallas as pl

_STD = (0.1, 0.1, 0.2, 0.2)
_MAX_RATIO = abs(float(np.log(16.0 / 1000.0)))


def _body(c, blk, y_ref, bb_ref, pr_ref, out_ref):
    y = y_ref[...]
    cls = jax.lax.broadcasted_iota(jnp.int32, (blk, c), 1)
    max_v = jnp.max(y, axis=-1, keepdims=True)
    label = jnp.min(jnp.where(y == max_v, cls, c), axis=-1, keepdims=True)

    bb = bb_ref[...]
    k = jax.lax.broadcasted_iota(jnp.int32, (blk, c * 4), 1)
    hit = (k // 4) == label
    comp = jax.lax.rem(k, 4)

    def pick(ci):
        m = hit & (comp == ci)
        return jnp.sum(jnp.where(m, bb, 0.0), axis=-1, keepdims=True)

    dx = pick(0) * _STD[0]
    dy = pick(1) * _STD[1]
    dw = jnp.clip(pick(2) * _STD[2], -_MAX_RATIO, _MAX_RATIO)
    dh = jnp.clip(pick(3) * _STD[3], -_MAX_RATIO, _MAX_RATIO)

    pr = pr_ref[...]
    x1 = pr[:, 0:1]
    y1 = pr[:, 1:2]
    w = pr[:, 2:3] - x1
    h = pr[:, 3:4] - y1
    cx = x1 + 0.5 * w
    cy = y1 + 0.5 * h
    pw = w * jnp.exp(dw)
    ph = h * jnp.exp(dh)
    pcx = cx + dx * w
    pcy = cy + dy * h

    o0 = pcx - 0.5 * pw
    o1 = pcy - 0.5 * ph
    o2 = pcx + 0.5 * pw
    o3 = pcy + 0.5 * ph
    kc = jax.lax.broadcasted_iota(jnp.int32, (blk, 4), 1)
    out = jnp.where(kc == 0, o0, jnp.where(kc == 1, o1, jnp.where(kc == 2, o2, o3)))
    out_ref[...] = jnp.clip(out, 0.0, 1.0)


def kernel(y_pred, bbox_pred, proposals):
    b, n, c = y_pred.shape
    r = b * n
    blk = 1000
    grid = r // blk
    y2 = y_pred.reshape(r, c)
    bb2 = bbox_pred.reshape(r, c * 4)
    pr2 = proposals.reshape(r, 4)
    out = pl.pallas_call(
        functools.partial(_body, c, blk),
        grid=(grid,),
        in_specs=[
            pl.BlockSpec((blk, c), lambda i: (i, 0)),
            pl.BlockSpec((blk, c * 4), lambda i: (i, 0)),
            pl.BlockSpec((blk, 4), lambda i: (i, 0)),
        ],
        out_specs=pl.BlockSpec((blk, 4), lambda i: (i, 0)),
        out_shape=jax.ShapeDtypeStruct((r, 4), jnp.float32),
    )(y2, bb2, pr2)
    return jax.lax.stop_gradient(out.reshape(b, n, 4))
